# baseline (speedup 1.0000x reference)
import functools

import jax
import jax.numpy as jnp
from jax import lax
from jax.experimental import pallas as pl
from jax.experimental.pallas import tpu as pltpu
from jax.experimental.pallas import tpu_sc as plsc

B, S = 4096, 200
D = 64
NC, NS = 2, 16
NW = NC * NS
CHUNK = 128
NCHUNK = S
G = 5
PITCH = CHUNK + 1


def _sc_gather(idx3, table):
    mesh = plsc.VectorSubcoreMesh(
        core_axis_name="c", subcore_axis_name="s",
        num_cores=NC, num_subcores=NS,
    )

    @functools.partial(
        pl.kernel,
        out_type=jax.ShapeDtypeStruct((S, 8, NW, 8, CHUNK), jnp.float32),
        mesh=mesh,
        compiler_params=pltpu.CompilerParams(
            use_tc_tiling_on_sc=False, needs_layout_passes=False),
        scratch_types=[
            pltpu.VMEM((NCHUNK, CHUNK), jnp.int32),
            pltpu.VMEM((G * CHUNK, D), jnp.float32),
            pltpu.VMEM((G * D, PITCH), jnp.float32),
            pltpu.SemaphoreType.DMA,
            pltpu.SemaphoreType.DMA,
        ],
    )
    def k(idx_hbm, table_hbm, out_hbm, idx_v, gbufs, tbufs, gsem, osem):
        wid = lax.axis_index("s") * NC + lax.axis_index("c")
        pltpu.sync_copy(idx_hbm.at[wid], idx_v)

        def fire_gather(c, g):
            pltpu.async_copy(
                table_hbm.at[idx_v.at[c]],
                gbufs.at[pl.ds(g * CHUNK, CHUNK)], gsem)

        def wait_gather():
            pltpu.make_async_copy(
                table_hbm.at[pl.ds(0, CHUNK)],
                gbufs.at[pl.ds(0, CHUNK)], gsem).wait()

        def wait_store():
            pltpu.make_async_copy(
                tbufs.at[pl.ds(0, 8), pl.ds(0, CHUNK)],
                out_hbm.at[0, 0, 0], osem).wait()

        for c in range(G):
            fire_gather(c, c)

        iota16 = lax.iota(jnp.int32, 16)
        rowv = [[iota16 + (g * D + 16 * kk) for kk in range(4)]
                for g in range(G)]

        @pl.loop(0, NCHUNK // G)
        def _(cs):
            c0 = cs * G
            for g in range(G):
                c = c0 + g
                @pl.when(c >= G)
                def _():
                    for _i in range(8):
                        wait_store()
                wait_gather()
                colv = jnp.full((16,), 0, jnp.int32)
                SKEW = 2

                def loads(b):
                    return [gbufs[g * CHUNK + b, pl.ds(16 * kk, 16)]
                            for kk in range(4)]

                pending = [loads(b) for b in range(SKEW)]
                for b in range(CHUNK):
                    if b + SKEW < CHUNK:
                        pending.append(loads(b + SKEW))
                    vs = pending.pop(0)
                    for kk in range(4):
                        plsc.store_scatter(tbufs, [rowv[g][kk], colv], vs[kk])
                    colv = colv + 1
                for rd in range(8):
                    pltpu.async_copy(
                        tbufs.at[pl.ds(g * D + 8 * rd, 8), pl.ds(0, CHUNK)],
                        out_hbm.at[c, rd, wid], osem)
                @pl.when(c + G < NCHUNK)
                def _():
                    fire_gather(c + G, g)

        for _i in range(G * 8):
            wait_store()

    return k(idx3, table)


def kernel(index, emb_weight):
    idx3 = index.T.reshape(S, NW, CHUNK).transpose(1, 0, 2)
    f2 = _sc_gather(idx3, emb_weight)
    return f2.transpose(2, 4, 0, 1, 3).reshape(B, S, D)


# speedup vs baseline: 1.1846x; 1.1846x over previous
import functools

import jax
import jax.numpy as jnp
from jax import lax
from jax.experimental import pallas as pl
from jax.experimental.pallas import tpu as pltpu
from jax.experimental.pallas import tpu_sc as plsc

B, S = 4096, 200
D = 64
NC, NS = 2, 16
NW = NC * NS
CHUNK = 128
NCHUNK = S
G = 8
T = 4
PITCH = CHUNK + 1
SKEW = 2


def _sc_gather(idx3, table):
    mesh = plsc.VectorSubcoreMesh(
        core_axis_name="c", subcore_axis_name="s",
        num_cores=NC, num_subcores=NS,
    )

    @functools.partial(
        pl.kernel,
        out_type=jax.ShapeDtypeStruct((S, 8, NW, 8, CHUNK), jnp.float32),
        mesh=mesh,
        compiler_params=pltpu.CompilerParams(
            use_tc_tiling_on_sc=False, needs_layout_passes=False),
        scratch_types=[
            pltpu.VMEM((NCHUNK, CHUNK), jnp.int32),
            pltpu.VMEM((G * CHUNK, D), jnp.float32),
            pltpu.VMEM((T, D, PITCH), jnp.float32),
            pltpu.SemaphoreType.DMA,
            pltpu.SemaphoreType.DMA,
        ],
    )
    def k(idx_hbm, table_hbm, out_hbm, idx_v, gbufs, tbufs, gsem, osem):
        wid = lax.axis_index("s") * NC + lax.axis_index("c")
        pltpu.sync_copy(idx_hbm.at[wid], idx_v)

        def fire_gather(c, g):
            pltpu.async_copy(
                table_hbm.at[idx_v.at[c]],
                gbufs.at[pl.ds(g * CHUNK, CHUNK)], gsem)

        def wait_gather():
            pltpu.make_async_copy(
                table_hbm.at[pl.ds(0, CHUNK)],
                gbufs.at[pl.ds(0, CHUNK)], gsem).wait()

        def wait_store():
            pltpu.make_async_copy(
                tbufs.at[0, pl.ds(0, 8), pl.ds(0, CHUNK)],
                out_hbm.at[0, 0, 0], osem).wait()

        for c in range(G):
            fire_gather(c, c)

        iota16 = lax.iota(jnp.int32, 16)
        dvec = [iota16 + 16 * k for k in range(4)]

        @pl.loop(0, NCHUNK)
        def _(c):
            @pl.when(c >= T)
            def _():
                for _i in range(8):
                    wait_store()
            wait_gather()
            g = lax.rem(c, G)
            t = lax.rem(c, T)
            gbase = g * CHUNK
            tsplat = jnp.full((16,), t, jnp.int32)

            def loads(b):
                return [gbufs[gbase + b, pl.ds(16 * kk, 16)]
                        for kk in range(4)]

            pending = [loads(b) for b in range(SKEW)]
            for b in range(CHUNK):
                if b + SKEW < CHUNK:
                    pending.append(loads(b + SKEW))
                vs = pending.pop(0)
                bsplat = jnp.full((16,), b, jnp.int32)
                for kk in range(4):
                    plsc.store_scatter(
                        tbufs, [tsplat, dvec[kk], bsplat], vs[kk])
            for rd in range(8):
                pltpu.async_copy(
                    tbufs.at[t, pl.ds(8 * rd, 8), pl.ds(0, CHUNK)],
                    out_hbm.at[c, rd, wid], osem)
            @pl.when(c + G < NCHUNK)
            def _():
                fire_gather(c + G, lax.rem(c + G, G))

        for _i in range(T * 8):
            wait_store()

    return k(idx3, table)


def kernel(index, emb_weight):
    idx3 = index.T.reshape(S, NW, CHUNK).transpose(1, 0, 2)
    f2 = _sc_gather(idx3, emb_weight)
    return f2.transpose(2, 4, 0, 1, 3).reshape(B, S, D)
